# baseline (speedup 1.0000x reference)
import jax
import jax.numpy as jnp
from jax import lax
from jax.experimental import pallas as pl
from jax.experimental.pallas import tpu as pltpu
from jax.experimental.pallas import tpu_sc as plsc

VOCAB = 100000
EMBED_DIM = 64
BATCH = 4096
SEQ = 50
NUM_INDICES = BATCH * SEQ

_info = plsc.get_sparse_core_info()
NC, NS = _info.num_cores, _info.num_subcores
NW = NC * NS
BPW = BATCH // NW
PER_W = BPW * SEQ
SCHUNK = 2
NCHUNK = SEQ // SCHUNK


def _embed_kernel(idx_hbm, table_hbm, out_hbm,
                  idx_all, idx2, rows, obuf, g_sems, w_sems):
    wid = lax.axis_index("s") * NC + lax.axis_index("c")
    base = wid * PER_W
    bbase = wid * BPW

    pltpu.sync_copy(idx_hbm.at[pl.ds(base, PER_W)], idx_all)

    iota = lax.iota(jnp.int32, 16)
    iota_seq = iota * SEQ

    def _reorder(s, _):
        for blg in range(BPW // 16):
            src = plsc.load_gather(idx_all, [iota_seq + (blg * 16 * SEQ + s)])
            idx2[pl.ds(s * BPW + blg * 16, 16)] = src
        return _
    lax.fori_loop(0, SEQ, _reorder, 0)

    gathers = [None] * NCHUNK
    writes = [None] * NCHUNK

    def _fire_gather(c):
        gathers[c] = pltpu.async_copy(
            table_hbm.at[idx2.at[pl.ds(c * SCHUNK * BPW, SCHUNK * BPW)]],
            rows.at[c % 2], g_sems.at[c % 2])

    _fire_gather(0)
    _fire_gather(1)

    for c in range(NCHUNK):
        b = c % 2
        gathers[c].wait()
        if c >= 2:
            writes[c - 2].wait()

        def _tr_token(sl, _):
            rowbase = iota + sl * BPW

            def _tr_d(d, __):
                dvec = jnp.full((16,), d, dtype=jnp.int32)
                for blg in range(BPW // 16):
                    v = plsc.load_gather(
                        rows.at[b], [rowbase + blg * 16, dvec])
                    obuf[b, sl, d, pl.ds(blg * 16, 16)] = v
                return __
            lax.fori_loop(0, EMBED_DIM, _tr_d, 0)
            return _
        lax.fori_loop(0, SCHUNK, _tr_token, 0)

        writes[c] = pltpu.async_copy(
            obuf.at[b],
            out_hbm.at[pl.ds(c * SCHUNK, SCHUNK), :, pl.ds(bbase, BPW)],
            w_sems.at[b])
        if c + 2 < NCHUNK:
            _fire_gather(c + 2)

    writes[NCHUNK - 2].wait()
    writes[NCHUNK - 1].wait()


@jax.jit
def _embed(idx_flat, weight):
    mesh = plsc.VectorSubcoreMesh(core_axis_name="c", subcore_axis_name="s")
    return pl.kernel(
        _embed_kernel,
        out_type=jax.ShapeDtypeStruct((SEQ, EMBED_DIM, BATCH), jnp.float32),
        mesh=mesh,
        scratch_types=[
            pltpu.VMEM((PER_W,), jnp.int32),
            pltpu.VMEM((PER_W,), jnp.int32),
            pltpu.VMEM((2, SCHUNK * BPW, EMBED_DIM), jnp.float32),
            pltpu.VMEM((2, SCHUNK, EMBED_DIM, BPW), jnp.float32),
            pltpu.SemaphoreType.DMA((2,)),
            pltpu.SemaphoreType.DMA((2,)),
        ],
        compiler_params=pltpu.CompilerParams(use_tc_tiling_on_sc=False,
                                             needs_layout_passes=False),
    )(idx_flat, weight)


def kernel(input_sentence, weight):
    idx_flat = input_sentence.reshape(-1).astype(jnp.int32)
    out_t = _embed(idx_flat, weight)
    return jnp.transpose(out_t, (2, 0, 1))


# speedup vs baseline: 1.1604x; 1.1604x over previous
import jax
import jax.numpy as jnp
from jax import lax
from jax.experimental import pallas as pl
from jax.experimental.pallas import tpu as pltpu
from jax.experimental.pallas import tpu_sc as plsc

VOCAB = 100000
EMBED_DIM = 64
BATCH = 4096
SEQ = 50
NUM_INDICES = BATCH * SEQ

_info = plsc.get_sparse_core_info()
NC, NS = _info.num_cores, _info.num_subcores
NW = NC * NS
BPW = BATCH // NW
PER_W = BPW * SEQ
NCHUNK = SEQ


def _embed_kernel(idx_hbm, table_hbm, out_hbm,
                  idx_all, idx2, rows, obuf, g_sems, w_sems):
    wid = lax.axis_index("s") * NC + lax.axis_index("c")
    base = wid * PER_W
    bbase = wid * BPW

    pltpu.sync_copy(idx_hbm.at[pl.ds(base, PER_W)], idx_all)

    iota = lax.iota(jnp.int32, 16)
    iota_seq = iota * SEQ

    def _reorder(s, carry):
        for blg in range(BPW // 16):
            src = plsc.load_gather(idx_all, [iota_seq + (blg * 16 * SEQ + s)])
            idx2[pl.ds(s * BPW + blg * 16, 16)] = src
        return carry
    lax.fori_loop(0, SEQ, _reorder, 0)

    drows = [iota + d0 for d0 in range(0, EMBED_DIM, 16)]

    gathers = [None] * NCHUNK
    writes = [None] * NCHUNK

    def _fire_gather(c):
        gathers[c] = pltpu.async_copy(
            table_hbm.at[idx2.at[pl.ds(c * BPW, BPW)]],
            rows.at[c % 2], g_sems.at[c % 2])

    _fire_gather(0)
    for c in range(NCHUNK):
        b = c % 2
        gathers[c].wait()
        if c + 1 < NCHUNK:
            _fire_gather(c + 1)
        if c >= 2:
            writes[c - 2].wait()

        def _tr(bl, carry):
            colv = jnp.full((16,), bl, dtype=jnp.int32)
            for j in range(EMBED_DIM // 16):
                v = rows[b, bl, pl.ds(j * 16, 16)]
                plsc.store_scatter(obuf.at[b], [drows[j], colv], v)
            return carry
        lax.fori_loop(0, BPW, _tr, 0)

        writes[c] = pltpu.async_copy(
            obuf.at[b], out_hbm.at[c, :, pl.ds(bbase, BPW)], w_sems.at[b])

    writes[NCHUNK - 2].wait()
    writes[NCHUNK - 1].wait()


@jax.jit
def _embed(idx_flat, weight):
    mesh = plsc.VectorSubcoreMesh(core_axis_name="c", subcore_axis_name="s")
    return pl.kernel(
        _embed_kernel,
        out_type=jax.ShapeDtypeStruct((SEQ, EMBED_DIM, BATCH), jnp.float32),
        mesh=mesh,
        scratch_types=[
            pltpu.VMEM((PER_W,), jnp.int32),
            pltpu.VMEM((PER_W,), jnp.int32),
            pltpu.VMEM((2, BPW, EMBED_DIM), jnp.float32),
            pltpu.VMEM((2, EMBED_DIM, BPW), jnp.float32),
            pltpu.SemaphoreType.DMA((2,)),
            pltpu.SemaphoreType.DMA((2,)),
        ],
        compiler_params=pltpu.CompilerParams(use_tc_tiling_on_sc=False,
                                             needs_layout_passes=False),
    )(idx_flat, weight)


def kernel(input_sentence, weight):
    idx_flat = input_sentence.reshape(-1).astype(jnp.int32)
    out_t = _embed(idx_flat, weight)
    return jnp.transpose(out_t, (2, 0, 1))


# speedup vs baseline: 1.4141x; 1.2186x over previous
import jax
import jax.numpy as jnp
from jax import lax
from jax.experimental import pallas as pl
from jax.experimental.pallas import tpu as pltpu
from jax.experimental.pallas import tpu_sc as plsc

VOCAB = 100000
EMBED_DIM = 64
BATCH = 4096
SEQ = 50
NUM_INDICES = BATCH * SEQ

_info = plsc.get_sparse_core_info()
NC, NS = _info.num_cores, _info.num_subcores
NW = NC * NS
BPW = BATCH // NW
PER_W = BPW * SEQ
NCHUNK = SEQ


def _embed_kernel(idx_hbm, table_hbm, out_hbm,
                  idx_all, idx2, rows, obuf, g_sems, w_sems):
    wid = lax.axis_index("s") * NC + lax.axis_index("c")
    base = wid * PER_W
    bbase = wid * BPW

    pltpu.sync_copy(idx_hbm.at[pl.ds(base, PER_W)], idx_all)

    iota = lax.iota(jnp.int32, 16)
    iota_seq = iota * SEQ

    def _reorder(s, carry):
        for blg in range(BPW // 16):
            src = plsc.load_gather(idx_all, [iota_seq + (blg * 16 * SEQ + s)])
            idx2[pl.ds(s * BPW + blg * 16, 16)] = src
        return carry
    lax.fori_loop(0, SEQ, _reorder, 0)

    drows = [iota + d0 for d0 in range(0, EMBED_DIM, 16)]

    gathers = [None] * NCHUNK
    writes = [None] * NCHUNK

    def _fire_gather(c):
        gathers[c] = pltpu.async_copy(
            table_hbm.at[idx2.at[pl.ds(c * BPW, BPW)]],
            rows.at[c % 2], g_sems.at[c % 2])

    _fire_gather(0)
    for c in range(NCHUNK):
        b = c % 2
        gathers[c].wait()
        if c + 1 < NCHUNK:
            _fire_gather(c + 1)
        if c >= 2:
            writes[c - 2].wait()

        @plsc.parallel_loop(0, BPW, unroll=4)
        def _tr(bl):
            colv = jnp.full((16,), bl, dtype=jnp.int32)
            for j in range(EMBED_DIM // 16):
                v = rows[b, bl, pl.ds(j * 16, 16)]
                plsc.store_scatter(obuf.at[b], [drows[j], colv], v)

        writes[c] = pltpu.async_copy(
            obuf.at[b], out_hbm.at[c, :, pl.ds(bbase, BPW)], w_sems.at[b])

    writes[NCHUNK - 2].wait()
    writes[NCHUNK - 1].wait()


@jax.jit
def _embed(idx_flat, weight):
    mesh = plsc.VectorSubcoreMesh(core_axis_name="c", subcore_axis_name="s")
    return pl.kernel(
        _embed_kernel,
        out_type=jax.ShapeDtypeStruct((SEQ, EMBED_DIM, BATCH), jnp.float32),
        mesh=mesh,
        scratch_types=[
            pltpu.VMEM((PER_W,), jnp.int32),
            pltpu.VMEM((PER_W,), jnp.int32),
            pltpu.VMEM((2, BPW, EMBED_DIM), jnp.float32),
            pltpu.VMEM((2, EMBED_DIM, BPW), jnp.float32),
            pltpu.SemaphoreType.DMA((2,)),
            pltpu.SemaphoreType.DMA((2,)),
        ],
        compiler_params=pltpu.CompilerParams(use_tc_tiling_on_sc=False,
                                             needs_layout_passes=False),
    )(idx_flat, weight)


def kernel(input_sentence, weight):
    idx_flat = input_sentence.reshape(-1).astype(jnp.int32)
    out_t = _embed(idx_flat, weight)
    return jnp.transpose(out_t, (2, 0, 1))
